# baseline (speedup 1.0000x reference)
import functools

import jax
import jax.numpy as jnp
from jax import lax
from jax.experimental import pallas as pl
from jax.experimental.pallas import tpu as pltpu
from jax.experimental.pallas import tpu_sc as plsc

_D = 64
_B = 4096 * 200

_info = plsc.get_sparse_core_info()
_NC, _NS = _info.num_cores, _info.num_subcores
_NW = _NC * _NS
_BPW = _B // _NW
_CHUNK = 1024
_NCHUNK = _BPW // _CHUNK
_KSUB = _CHUNK // 128


def _sc_gather(idx_hbm, table_hbm, out_hbm, idx_v, rows_v, sem_idx, sem_g,
               sem_out):
    wid = lax.axis_index("s") * _NC + lax.axis_index("c")
    base = wid * _BPW

    def chunk(i, _):
        off = pl.multiple_of(base + i * _CHUNK, _CHUNK)
        pltpu.async_copy(idx_hbm.at[pl.ds(off, _CHUNK)], idx_v, sem_idx).wait()
        cps = []
        for j in range(_KSUB):
            cps.append(pltpu.async_copy(
                table_hbm.at[idx_v.at[pl.ds(j * 128, 128)]],
                rows_v.at[pl.ds(j * 128, 128)], sem_g))
        for cp in cps:
            cp.wait()
        pltpu.async_copy(rows_v, out_hbm.at[pl.ds(off, _CHUNK)], sem_out).wait()
        return ()

    lax.fori_loop(0, _NCHUNK, chunk, ())


@jax.jit
def _embed(token_ids_flat, weight):
    mesh = plsc.VectorSubcoreMesh(core_axis_name="c", subcore_axis_name="s")
    k = functools.partial(
        pl.kernel,
        mesh=mesh,
        compiler_params=pltpu.CompilerParams(use_tc_tiling_on_sc=False),
        out_type=jax.ShapeDtypeStruct((_B, _D), jnp.float32),
        scratch_types=[
            pltpu.VMEM((_CHUNK,), jnp.int32),
            pltpu.VMEM((_CHUNK, _D), jnp.float32),
            pltpu.SemaphoreType.DMA,
            pltpu.SemaphoreType.DMA,
            pltpu.SemaphoreType.DMA,
        ],
    )(_sc_gather)
    return k(token_ids_flat, weight)


def kernel(token_ids, weight):
    flat = token_ids.reshape(-1).astype(jnp.int32)
    out = _embed(flat, weight)
    return out.reshape(token_ids.shape + (weight.shape[1],))


# speedup vs baseline: 1.0092x; 1.0092x over previous
import functools

import jax
import jax.numpy as jnp
from jax import lax
from jax.experimental import pallas as pl
from jax.experimental.pallas import tpu as pltpu
from jax.experimental.pallas import tpu_sc as plsc

_D = 64
_B = 4096 * 200

_info = plsc.get_sparse_core_info()
_NC, _NS = _info.num_cores, _info.num_subcores
_NW = _NC * _NS
_BPW = _B // _NW
_CHUNK = 640
_NCHUNK = _BPW // _CHUNK
_KSUB = _CHUNK // 128


def _sc_gather(idx_hbm, table_hbm, out_hbm, idx0, idx1, rows0, rows1,
               sem_idx, sem_g, semo0, semo1):
    wid = lax.axis_index("s") * _NC + lax.axis_index("c")
    base = wid * _BPW
    idx_v = (idx0, idx1)
    rows_v = (rows0, rows1)
    sem_out = (semo0, semo1)

    def pair(g, _):
        for s in range(2):
            off = pl.multiple_of(base + (g * 2 + s) * _CHUNK, _CHUNK)
            pltpu.async_copy(idx_hbm.at[pl.ds(off, _CHUNK)], idx_v[s],
                             sem_idx).wait()

            @pl.when(g > 0)
            def _():
                pltpu.make_async_copy(rows_v[s], out_hbm.at[pl.ds(off, _CHUNK)],
                                      sem_out[s]).wait()

            cps = []
            for j in range(_KSUB):
                cps.append(pltpu.async_copy(
                    table_hbm.at[idx_v[s].at[pl.ds(j * 128, 128)]],
                    rows_v[s].at[pl.ds(j * 128, 128)], sem_g))
            for cp in cps:
                cp.wait()
            pltpu.async_copy(rows_v[s], out_hbm.at[pl.ds(off, _CHUNK)],
                             sem_out[s])
        return ()

    lax.fori_loop(0, _NCHUNK // 2, pair, ())
    for s in range(2):
        off = pl.multiple_of(base + (_NCHUNK - 2 + s) * _CHUNK, _CHUNK)
        pltpu.make_async_copy(rows_v[s], out_hbm.at[pl.ds(off, _CHUNK)],
                              sem_out[s]).wait()


@jax.jit
def _embed(token_ids_flat, weight):
    mesh = plsc.VectorSubcoreMesh(core_axis_name="c", subcore_axis_name="s")
    k = functools.partial(
        pl.kernel,
        mesh=mesh,
        compiler_params=pltpu.CompilerParams(use_tc_tiling_on_sc=False),
        out_type=jax.ShapeDtypeStruct((_B, _D), jnp.float32),
        scratch_types=[
            pltpu.VMEM((_CHUNK,), jnp.int32),
            pltpu.VMEM((_CHUNK,), jnp.int32),
            pltpu.VMEM((_CHUNK, _D), jnp.float32),
            pltpu.VMEM((_CHUNK, _D), jnp.float32),
            pltpu.SemaphoreType.DMA,
            pltpu.SemaphoreType.DMA,
            pltpu.SemaphoreType.DMA,
            pltpu.SemaphoreType.DMA,
        ],
    )(_sc_gather)
    return k(token_ids_flat, weight)


def kernel(token_ids, weight):
    flat = token_ids.reshape(-1).astype(jnp.int32)
    out = _embed(flat, weight)
    return out.reshape(token_ids.shape + (weight.shape[1],))
